# baseline (speedup 1.0000x reference)
import jax
import jax.numpy as jnp
from jax.experimental import pallas as pl
from jax.experimental.pallas import tpu as pltpu


def _matcher_kernel(ts_ref, te_ref, es_ref, ee_ref, out_ref, vmax_ref, vidx_ref):
    i = pl.program_id(0)
    h = i % 2
    ws = jnp.exp(ts_ref[0])
    we = jnp.exp(te_ref[0])
    es = es_ref[0]
    ee = ee_ref[0]
    dn = (((1,), (1,)), ((), ()))
    st = jax.lax.dot_general(es, ws, dn, precision=jax.lax.Precision.HIGHEST,
                             preferred_element_type=jnp.float32)
    st = st + jax.lax.dot_general(ee, we, dn, precision=jax.lax.Precision.HIGHEST,
                                  preferred_element_type=jnp.float32)
    mx = jnp.max(st, axis=0, keepdims=True)
    iota = jax.lax.broadcasted_iota(jnp.int32, st.shape, 0)
    idx = jnp.min(jnp.where(st == mx, iota, 2**30), axis=0, keepdims=True)
    idx = idx + h * st.shape[0]

    @pl.when(h == 0)
    def _init():
        vmax_ref[...] = mx
        vidx_ref[...] = idx

    @pl.when(h == 1)
    def _merge():
        better = mx > vmax_ref[...]
        out_ref[0, 0, :] = jnp.where(better, idx, vidx_ref[...])[0]


def kernel(ent_start_probs, ent_end_probs, ent_part_probs,
           target_start_probs, target_end_probs, target_part_probs):
    B, N, L = ent_start_probs.shape
    M = target_start_probs.shape[1]
    nh = N // 2
    es2 = ent_start_probs.reshape(2 * B, nh, L)
    ee2 = ent_end_probs.reshape(2 * B, nh, L)
    out = pl.pallas_call(
        _matcher_kernel,
        grid=(2 * B,),
        in_specs=[
            pl.BlockSpec((1, M, L), lambda i: (i // 2, 0, 0)),
            pl.BlockSpec((1, M, L), lambda i: (i // 2, 0, 0)),
            pl.BlockSpec((1, nh, L), lambda i: (i, 0, 0)),
            pl.BlockSpec((1, nh, L), lambda i: (i, 0, 0)),
        ],
        out_specs=pl.BlockSpec((1, 1, M), lambda i: (i // 2, 0, 0)),
        out_shape=jax.ShapeDtypeStruct((B, 1, M), jnp.int32),
        scratch_shapes=[
            pltpu.VMEM((1, M), jnp.float32),
            pltpu.VMEM((1, M), jnp.int32),
        ],
    )(target_start_probs, target_end_probs, es2, ee2)
    return out.reshape(B, M)


# speedup vs baseline: 1.1564x; 1.1564x over previous
import functools

import jax
import jax.numpy as jnp
from jax.experimental import pallas as pl


def _matcher_kernel(m, n, ts_ref, te_ref, es_ref, ee_ref, out_ref):
    ws = jnp.exp(ts_ref[...])
    we = jnp.exp(te_ref[...])
    es = es_ref[...]
    ee = ee_ref[...]
    dn = (((1,), (1,)), ((), ()))
    st = jax.lax.dot_general(es, ws, dn, precision=jax.lax.Precision.HIGHEST,
                             preferred_element_type=jnp.float32)
    st = st + jax.lax.dot_general(ee, we, dn, precision=jax.lax.Precision.HIGHEST,
                                  preferred_element_type=jnp.float32)
    rown = jax.lax.broadcasted_iota(jnp.int32, st.shape, 0)
    colm = jax.lax.broadcasted_iota(jnp.int32, st.shape, 1)
    valid = (rown // n) == (colm // m)
    st = jnp.where(valid, st, -jnp.inf)
    mx = jnp.max(st, axis=0, keepdims=True)
    idx = jnp.min(jnp.where(st == mx, rown, 2**30), axis=0, keepdims=True)
    col = jax.lax.broadcasted_iota(jnp.int32, idx.shape, 1)
    out_ref[...] = idx - (col // m) * n


def kernel(ent_start_probs, ent_end_probs, ent_part_probs,
           target_start_probs, target_end_probs, target_part_probs):
    B, N, L = ent_start_probs.shape
    M = target_start_probs.shape[1]
    out = pl.pallas_call(
        functools.partial(_matcher_kernel, M, N),
        out_shape=jax.ShapeDtypeStruct((1, B * M), jnp.int32),
    )(target_start_probs.reshape(B * M, L), target_end_probs.reshape(B * M, L),
      ent_start_probs.reshape(B * N, L), ent_end_probs.reshape(B * N, L))
    return out.reshape(B, M)


# speedup vs baseline: 1.5129x; 1.3083x over previous
import jax
import jax.numpy as jnp
from jax.experimental import pallas as pl
from jax.experimental.pallas import tpu as pltpu


def _matcher_kernel(ts_ref, te_ref, es_ref, ee_ref, out_ref):
    ws = jnp.exp(ts_ref[0])
    we = jnp.exp(te_ref[0])
    es = es_ref[0]
    ee = ee_ref[0]
    dn = (((1,), (1,)), ((), ()))
    st = jax.lax.dot_general(es, ws, dn, precision=jax.lax.Precision.HIGHEST,
                             preferred_element_type=jnp.float32)
    st = st + jax.lax.dot_general(ee, we, dn, precision=jax.lax.Precision.HIGHEST,
                                  preferred_element_type=jnp.float32)
    mx = jnp.max(st, axis=0, keepdims=True)
    iota = jax.lax.broadcasted_iota(jnp.int32, st.shape, 0)
    idx = jnp.min(jnp.where(st == mx, iota, 2**30), axis=0)
    out_ref[0, 0, :] = idx


def kernel(ent_start_probs, ent_end_probs, ent_part_probs,
           target_start_probs, target_end_probs, target_part_probs):
    B, N, L = ent_start_probs.shape
    M = target_start_probs.shape[1]
    out = pl.pallas_call(
        _matcher_kernel,
        grid=(B,),
        in_specs=[
            pl.BlockSpec((1, M, L), lambda i: (i, 0, 0)),
            pl.BlockSpec((1, M, L), lambda i: (i, 0, 0)),
            pl.BlockSpec((1, N, L), lambda i: (i, 0, 0)),
            pl.BlockSpec((1, N, L), lambda i: (i, 0, 0)),
        ],
        out_specs=pl.BlockSpec((1, 1, M), lambda i: (i, 0, 0)),
        out_shape=jax.ShapeDtypeStruct((B, 1, M), jnp.int32),
        compiler_params=pltpu.CompilerParams(dimension_semantics=("parallel",)),
    )(target_start_probs, target_end_probs, ent_start_probs, ent_end_probs)
    return out.reshape(B, M)


# speedup vs baseline: 1.5166x; 1.0024x over previous
import jax
import jax.numpy as jnp
from jax.experimental import pallas as pl
from jax.experimental.pallas import tpu as pltpu


def _matcher_kernel(ts_ref, te_ref, es_ref, ee_ref, out_ref):
    ws = jnp.exp(ts_ref[0])
    we = jnp.exp(te_ref[0])
    es = es_ref[0]
    ee = ee_ref[0]
    dn = (((1,), (1,)), ((), ()))
    st = jax.lax.dot_general(es, ws, dn, precision=jax.lax.Precision.HIGHEST,
                             preferred_element_type=jnp.float32)
    st = st + jax.lax.dot_general(ee, we, dn, precision=jax.lax.Precision.HIGHEST,
                                  preferred_element_type=jnp.float32)
    mx = jnp.max(st, axis=0, keepdims=True)
    iota = jax.lax.broadcasted_iota(jnp.int32, st.shape, 0)
    idx = jnp.min(jnp.where(st == mx, iota, 2**30), axis=0)
    out_ref[0, 0, :] = idx


def kernel(ent_start_probs, ent_end_probs, ent_part_probs,
           target_start_probs, target_end_probs, target_part_probs):
    B, N, L = ent_start_probs.shape
    M = target_start_probs.shape[1]
    out = pl.pallas_call(
        _matcher_kernel,
        grid=(B,),
        in_specs=[
            pl.BlockSpec((1, M, L), lambda i: (i, 0, 0)),
            pl.BlockSpec((1, M, L), lambda i: (i, 0, 0)),
            pl.BlockSpec((1, N, L), lambda i: (i, 0, 0)),
            pl.BlockSpec((1, N, L), lambda i: (i, 0, 0)),
        ],
        out_specs=pl.BlockSpec((1, 1, M), lambda i: (i, 0, 0)),
        out_shape=jax.ShapeDtypeStruct((B, 1, M), jnp.int32),
        compiler_params=pltpu.CompilerParams(dimension_semantics=("parallel",),
                                             disable_bounds_checks=True),
    )(target_start_probs, target_end_probs, ent_start_probs, ent_end_probs)
    return out.reshape(B, M)
